# baseline (speedup 1.0000x reference)
import functools

import jax
import jax.numpy as jnp
from jax import lax
from jax.experimental import pallas as pl
from jax.experimental.pallas import tpu as pltpu
from jax.experimental.pallas import tpu_sc as plsc

VOCAB = 1024
BLOCK_ROWS = 2048

N_ROWS = 16384
NC, NS, L = 2, 16, 16
NW = NC * NS
ROWS_PER_W = N_ROWS // NW
N_CHUNKS = ROWS_PER_W // L


def _hidden_body(ids_ref, hidden_ref):
    ids = ids_ref[...]
    iota = lax.broadcasted_iota(jnp.int32, (BLOCK_ROWS, VOCAB), 1)
    hidden_ref[...] = jnp.where(iota == ids, 1.0, 0.0)


_sc_mesh = plsc.VectorSubcoreMesh(
    core_axis_name="c", subcore_axis_name="s", num_cores=NC, num_subcores=NS
)


@functools.partial(
    pl.kernel,
    out_type=jax.ShapeDtypeStruct((N_ROWS, VOCAB), jnp.float32),
    mesh=_sc_mesh,
    scratch_types=[
        pltpu.VMEM((ROWS_PER_W,), jnp.int32),
        pltpu.VMEM((3 * L, VOCAB), jnp.float32),
        pltpu.VMEM((3, ROWS_PER_W), jnp.int32),
        pltpu.SemaphoreType.DMA,
    ],
    compiler_params=pltpu.CompilerParams(needs_layout_passes=False),
)
def _sc_logits(ids_hbm, out_hbm, ids_v, bufs, rowidx, sem):
    wid = lax.axis_index("s") * NC + lax.axis_index("c")
    base = wid * ROWS_PER_W
    pltpu.sync_copy(ids_hbm.at[pl.ds(base, ROWS_PER_W)], ids_v)

    iota16 = lax.broadcasted_iota(jnp.int32, (L,), 0)
    for jj in range(3):
        poke = jnp.where(iota16 == jj + 1, 50.0, -50.0)
        fill = jnp.full((L,), -50.0, jnp.float32)

        def fill_row(r, _, jj=jj, poke=poke, fill=fill):
            for k in range(VOCAB // L):
                bufs[jj * L + r, pl.ds(k * L, L)] = poke if k == 0 else fill
            return 0

        lax.fori_loop(0, L, fill_row, 0)

    def compact(c, carry):
        cnts, lasts = carry
        idvec = ids_v[pl.ds(c * L, L)]
        rowvec = base + c * L + iota16
        cls = lax.rem(idvec + 2, 3)
        new_cnts, new_lasts = [], []
        for jj in range(3):
            m = cls == jj
            mi = m.astype(jnp.int32)
            pos = cnts[jj] + plsc.cumsum(mi) - 1
            plsc.store_scatter(
                rowidx, [jnp.full((L,), jj, jnp.int32), pos], rowvec, mask=m
            )
            new_cnts.append(cnts[jj] + jnp.sum(mi))
            new_lasts.append(
                jnp.maximum(lasts[jj], jnp.max(jnp.where(m, rowvec, -1)))
            )
        return tuple(new_cnts), tuple(new_lasts)

    zero = jnp.int32(0)
    neg = jnp.int32(-1)
    (cnts, lasts) = lax.fori_loop(
        0, N_CHUNKS, compact, ((zero, zero, zero), (neg, neg, neg))
    )

    total = zero
    for jj in range(3):
        cnt, last = cnts[jj], lasts[jj]
        nch = (cnt + (L - 1)) // L

        def fire(c, _, jj=jj, cnt=cnt, last=last):
            w = rowidx[jj, pl.ds(c * L, L)]
            idx = jnp.where(iota16 < cnt - c * L, w, jnp.full((L,), last))
            pltpu.async_copy(
                bufs.at[pl.ds(jj * L, L)], out_hbm.at[idx], sem
            )
            return 0

        lax.fori_loop(0, nch, fire, 0)
        total = total + nch

    def drain(c, _):
        pltpu.make_async_copy(
            out_hbm.at[pl.ds(0, L)], bufs.at[pl.ds(0, L)], sem
        ).wait()
        return 0

    lax.fori_loop(0, total, drain, 0)


def kernel(input_ids, output_hidden_states):
    bsz, seq = input_ids.shape
    n = bsz * seq
    ids2d = input_ids.reshape(n, 1)
    hidden = pl.pallas_call(
        _hidden_body,
        grid=(n // BLOCK_ROWS,),
        in_specs=[pl.BlockSpec((BLOCK_ROWS, 1), lambda i: (i, 0))],
        out_specs=pl.BlockSpec((BLOCK_ROWS, VOCAB), lambda i: (i, 0)),
        out_shape=jax.ShapeDtypeStruct((n, VOCAB), jnp.float32),
        compiler_params=pltpu.CompilerParams(
            dimension_semantics=("arbitrary",),
        ),
    )(ids2d)
    ids_flat = input_ids.reshape(n)
    logits = _sc_logits(ids_flat)
    return (logits.reshape(bsz, seq, VOCAB), hidden.reshape(bsz, seq, VOCAB))


# speedup vs baseline: 1.3760x; 1.3760x over previous
import jax
import jax.numpy as jnp
from jax.experimental import pallas as pl
from jax.experimental.pallas import tpu as pltpu

VOCAB = 1024
BLOCK_ROWS = 1024


def _body(ids_ref, logits_ref, hidden_ref):
    ids = ids_ref[...]
    iota = jax.lax.broadcasted_iota(jnp.int32, (BLOCK_ROWS, VOCAB), 1)
    pred = jax.lax.rem(ids + 2, 3) + 1
    logits_ref[...] = jnp.where(iota == pred, 50.0, -50.0)
    hidden_ref[...] = jnp.where(iota == ids, 1.0, 0.0)


def kernel(input_ids, output_hidden_states):
    bsz, seq = input_ids.shape
    n = bsz * seq
    ids2d = input_ids.reshape(n, 1)
    grid = n // BLOCK_ROWS
    out_shape = [
        jax.ShapeDtypeStruct((n, VOCAB), jnp.float32),
        jax.ShapeDtypeStruct((n, VOCAB), jnp.float32),
    ]
    logits, hidden = pl.pallas_call(
        _body,
        grid=(grid,),
        in_specs=[pl.BlockSpec((BLOCK_ROWS, 1), lambda i: (i, 0))],
        out_specs=[
            pl.BlockSpec((BLOCK_ROWS, VOCAB), lambda i: (i, 0)),
            pl.BlockSpec((BLOCK_ROWS, VOCAB), lambda i: (i, 0)),
        ],
        out_shape=out_shape,
        compiler_params=pltpu.CompilerParams(
            dimension_semantics=("arbitrary",),
        ),
    )(ids2d)
    logits = logits.reshape(bsz, seq, VOCAB)
    hidden = hidden.reshape(bsz, seq, VOCAB)
    return (logits, hidden)


# speedup vs baseline: 1.3829x; 1.0050x over previous
import jax
import jax.numpy as jnp
from jax.experimental import pallas as pl
from jax.experimental.pallas import tpu as pltpu

VOCAB = 1024
BLOCK_ROWS = 1024


def _body(ids_ref, logits_ref, hidden_ref):
    ids = ids_ref[...]
    iota = jax.lax.broadcasted_iota(jnp.int32, (BLOCK_ROWS, VOCAB), 1)
    pred = jax.lax.rem(ids + 2, 3) + 1
    logits_ref[...] = jnp.where(iota == pred, 50.0, -50.0)
    hidden_ref[...] = jnp.where(iota == ids, 1.0, 0.0)


def kernel(input_ids, output_hidden_states):
    bsz, seq = input_ids.shape
    n = bsz * seq
    ids2d = input_ids.reshape(n, 1)
    grid = n // BLOCK_ROWS
    out_shape = [
        jax.ShapeDtypeStruct((n, VOCAB), jnp.float32),
        jax.ShapeDtypeStruct((n, VOCAB), jnp.float32),
    ]
    logits, hidden = pl.pallas_call(
        _body,
        grid=(grid,),
        in_specs=[pl.BlockSpec((BLOCK_ROWS, 1), lambda i: (i, 0))],
        out_specs=[
            pl.BlockSpec((BLOCK_ROWS, VOCAB), lambda i: (i, 0)),
            pl.BlockSpec((BLOCK_ROWS, VOCAB), lambda i: (i, 0)),
        ],
        out_shape=out_shape,
        compiler_params=pltpu.CompilerParams(
            dimension_semantics=("parallel",),
        ),
    )(ids2d)
    logits = logits.reshape(bsz, seq, VOCAB)
    hidden = hidden.reshape(bsz, seq, VOCAB)
    return (logits, hidden)
